# baseline (speedup 1.0000x reference)
import functools

import jax
import jax.numpy as jnp
from jax import lax
from jax.experimental import pallas as pl
from jax.experimental.pallas import tpu as pltpu
from jax.experimental.pallas import tpu_sc as plsc

DIM = 128
BATCH = 16384
EPS = 1e-5

_NSPLIT = 4
_BCH = BATCH // _NSPLIT

_NC = 2
_NS = 16
_NW = _NC * _NS
_ROWS_PER_W = _BCH // _NW
_CHUNK = 128
_NCHUNK = _ROWS_PER_W // _CHUNK


def _sc_gather(k, uid, iid, gid, cid, utab, itab, gtab, ctab):
    mesh = plsc.VectorSubcoreMesh(
        core_axis_name="c", subcore_axis_name="s",
        num_cores=_NC, num_subcores=_NS)

    @functools.partial(
        pl.kernel,
        out_type=[jax.ShapeDtypeStruct((_BCH, DIM), jnp.float32)] * 4,
        mesh=mesh,
        scratch_types=[
            pltpu.VMEM((4, _ROWS_PER_W), jnp.int32),
            [pltpu.VMEM((_CHUNK, DIM), jnp.float32)] * (4 * _NCHUNK),
            [pltpu.SemaphoreType.DMA] * 4,
            [pltpu.SemaphoreType.DMA] * (4 * _NCHUNK),
            [pltpu.SemaphoreType.DMA] * (4 * _NCHUNK),
        ],
    )
    def gather_kernel(uid_h, iid_h, gid_h, cid_h, ut_h, it_h, gt_h, ct_h,
                      o0, o1, o2, o3, idx_v, bufs, isems, gsems, wsems):
        wid = lax.axis_index("s") * _NC + lax.axis_index("c")
        base = wid * _ROWS_PER_W
        idx_hs = (uid_h, iid_h, gid_h, cid_h)
        tab_hs = (ut_h, it_h, gt_h, ct_h)
        out_hs = (o0, o1, o2, o3)
        ic = [pltpu.async_copy(
                  idx_hs[t].at[pl.ds(k * _BCH + base, _ROWS_PER_W)],
                  idx_v.at[t], isems[t]) for t in range(4)]
        g = [None] * (4 * _NCHUNK)
        wb = [None] * (4 * _NCHUNK)
        for t in range(4):
            ic[t].wait()
            for c in range(_NCHUNK):
                i = t * _NCHUNK + c
                g[i] = pltpu.async_copy(
                    tab_hs[t].at[idx_v.at[t, pl.ds(c * _CHUNK, _CHUNK)]],
                    bufs[i], gsems[i])
        for t in range(4):
            for c in range(_NCHUNK):
                i = t * _NCHUNK + c
                g[i].wait()
                wb[i] = pltpu.async_copy(
                    bufs[i],
                    out_hs[t].at[pl.ds(base + c * _CHUNK, _CHUNK)],
                    wsems[i])
        for i in range(4 * _NCHUNK):
            wb[i].wait()

    return gather_kernel(uid, iid, gid, cid, utab, itab, gtab, ctab)


_TB = 4096
_TB_OUT = 4096


def _moments_body(e0, e1, e2, e3, w1, hin, s_out, m_out, h_out):
    i = pl.program_id(0)
    h = (jnp.dot(e0[...], w1[0:128, :], preferred_element_type=jnp.float32)
         + jnp.dot(e1[...], w1[128:256, :], preferred_element_type=jnp.float32)
         + jnp.dot(e2[...], w1[256:384, :], preferred_element_type=jnp.float32)
         + jnp.dot(e3[...], w1[384:512, :], preferred_element_type=jnp.float32))

    @pl.when(i == 0)
    def _init():
        s_out[...] = jnp.zeros_like(s_out)
        m_out[...] = jnp.zeros_like(m_out)

    s_out[...] += jnp.sum(h, axis=0, keepdims=True)
    m_out[...] += lax.dot_general(h, h, (((0,), (0,)), ((), ())),
                                  preferred_element_type=jnp.float32)
    h_out[...] = h.astype(jnp.bfloat16)


def _moments(k, e0, e1, e2, e3, w1, hbuf):
    grid = (_BCH // _TB,)
    nblk = _BCH // _TB
    eb = pl.BlockSpec((_TB, DIM), lambda i: (i, 0))
    body = _moments_body if k > 0 else (
        lambda e0, e1, e2, e3, w1, s, m, h:
            _moments_body(e0, e1, e2, e3, w1, None, s, m, h))
    return pl.pallas_call(
        body,
        grid=grid,
        in_specs=[eb, eb, eb, eb, pl.BlockSpec((512, 256), lambda i: (0, 0))] +
                 ([pl.BlockSpec(memory_space=pltpu.MemorySpace.HBM)]
                  if k > 0 else []),
        out_specs=[pl.BlockSpec((1, 256), lambda i: (0, 0)),
                   pl.BlockSpec((256, 256), lambda i: (0, 0)),
                   pl.BlockSpec((_TB, 256), lambda i: (k * nblk + i, 0))],
        out_shape=[jax.ShapeDtypeStruct((1, 256), jnp.float32),
                   jax.ShapeDtypeStruct((256, 256), jnp.float32),
                   jax.ShapeDtypeStruct((BATCH, 256), jnp.bfloat16)],
        input_output_aliases={5: 2} if k > 0 else {},
    )(e0, e1, e2, e3, w1, *([hbuf] if k > 0 else []))


def _col(row):
    k = row.shape[1]
    b = jnp.broadcast_to(row, (k, k))
    rows_i = lax.broadcasted_iota(jnp.int32, (k, k), 0)
    cols_i = lax.broadcasted_iota(jnp.int32, (k, k), 1)
    return jnp.sum(jnp.where(rows_i == cols_i, b, 0.0), axis=1, keepdims=True)


def _chain_math(s1, m1, g1, b1, w2, g2, b2, w3, g3, b3):
    n = float(BATCH)
    mu1 = s1 / n
    m1n = m1 / n
    mu1c = _col(mu1)
    cov1 = m1n - mu1c * mu1
    rows_i = lax.broadcasted_iota(jnp.int32, (256, 256), 0)
    cols_i = lax.broadcasted_iota(jnp.int32, (256, 256), 1)
    var1 = jnp.sum(jnp.where(rows_i == cols_i, cov1, 0.0),
                   axis=0, keepdims=True)
    a1 = g1 * lax.rsqrt(var1 + EPS)
    c1 = b1 - mu1 * a1

    a1c = _col(a1)
    cov1s = cov1 * (a1c * a1)
    t2 = jnp.dot(cov1s, w2, preferred_element_type=jnp.float32)
    var2 = jnp.sum(w2 * t2, axis=0, keepdims=True)
    mu2 = jnp.dot(mu1 * a1 + c1, w2,
                  preferred_element_type=jnp.float32)
    a2 = g2 * lax.rsqrt(var2 + EPS)
    c2 = b2 - mu2 * a2

    cov2 = lax.dot_general(w2, t2, (((0,), (0,)), ((), ())),
                           preferred_element_type=jnp.float32)
    a2c = _col(a2)
    cov2s = cov2 * (a2c * a2)
    t3 = jnp.dot(cov2s, w3, preferred_element_type=jnp.float32)
    var3 = jnp.sum(w3 * t3, axis=0, keepdims=True)
    mu3 = jnp.dot(mu2 * a2 + c2, w3,
                  preferred_element_type=jnp.float32)
    a3 = g3 * lax.rsqrt(var3 + EPS)
    c3 = b3 - mu3 * a3

    r12 = (a1c * w2) * a2
    r = jnp.dot(r12, w3,
                preferred_element_type=jnp.float32) * a3
    q12 = jnp.dot(c1, w2, preferred_element_type=jnp.float32) * a2 + c2
    q = jnp.dot(q12, w3, preferred_element_type=jnp.float32) * a3 + c3
    return r, _col(q)


def _output_body(*refs):
    h1 = refs[0]
    s_refs = refs[1:1 + _NSPLIT]
    m_refs = refs[1 + _NSPLIT:1 + 2 * _NSPLIT]
    (g1, b1, w2, g2, b2, w3, g3, b3, o, r_v, q_v) = refs[1 + 2 * _NSPLIT:]
    i = pl.program_id(0)

    @pl.when(i == 0)
    def _fold():
        s1 = sum(ref[...] for ref in s_refs)
        m1 = sum(ref[...] for ref in m_refs)
        r, qt = _chain_math(s1, m1, g1[...], b1[...], w2[...],
                            g2[...], b2[...], w3[...], g3[...], b3[...])
        r_v[...] = r
        q_v[...] = qt

    acc = lax.dot_general(r_v[...], h1[...].astype(jnp.float32),
                          (((0,), (1,)), ((), ())),
                          preferred_element_type=jnp.float32)
    o[...] = jax.nn.sigmoid(acc + q_v[...])


def _output(h1, s_parts, m_parts, g1, b1, w2, g2, b2, w3, g3, b3):
    grid = (BATCH // _TB_OUT,)
    full = lambda shape: pl.BlockSpec(shape, lambda i: (0, 0))
    return pl.pallas_call(
        _output_body,
        grid=grid,
        in_specs=[pl.BlockSpec((_TB_OUT, 256), lambda i: (i, 0))] +
                 [full((1, 256))] * _NSPLIT + [full((256, 256))] * _NSPLIT +
                 [full((1, 256)), full((1, 256)), full((256, 128)),
                  full((1, 128)), full((1, 128)), full((128, 2)),
                  full((1, 2)), full((1, 2))],
        out_specs=pl.BlockSpec((2, _TB_OUT), lambda i: (0, i)),
        out_shape=jax.ShapeDtypeStruct((2, BATCH), jnp.float32),
        scratch_shapes=[pltpu.VMEM((256, 2), jnp.float32),
                        pltpu.VMEM((2, 1), jnp.float32)],
    )(h1, *s_parts, *m_parts, g1, b1, w2, g2, b2, w3, g3, b3)


def kernel(user_id, item_id, user_geohash, item_cate, label,
           user_tab, item_tab, geo_tab, cate_tab,
           W1, g1, b1, W2, g2, b2, W3, g3, b3):
    s_parts, m_parts = [], []
    hbuf = None
    for k in range(_NSPLIT):
        e0, e1, e2, e3 = _sc_gather(k, user_id, item_id, user_geohash,
                                    item_cate, user_tab, item_tab,
                                    geo_tab, cate_tab)
        s1, m1, hbuf = _moments(k, e0, e1, e2, e3, W1, hbuf)
        s_parts.append(s1)
        m_parts.append(m1)
    return jnp.transpose(_output(
        hbuf, s_parts, m_parts,
        g1.reshape(1, 256), b1.reshape(1, 256), W2,
        g2.reshape(1, 128), b2.reshape(1, 128), W3,
        g3.reshape(1, 2), b3.reshape(1, 2)))


# speedup vs baseline: 1.0190x; 1.0190x over previous
import functools

import jax
import jax.numpy as jnp
from jax import lax
from jax.experimental import pallas as pl
from jax.experimental.pallas import tpu as pltpu
from jax.experimental.pallas import tpu_sc as plsc

DIM = 128
BATCH = 16384
EPS = 1e-5

_NSPLIT = 4
_BCH = BATCH // _NSPLIT

_NC = 2
_NS = 16
_NW = _NC * _NS
_ROWS_PER_W = _BCH // _NW
_CHUNK = 128
_NCHUNK = _ROWS_PER_W // _CHUNK


def _sc_gather(k, uid, iid, gid, cid, utab, itab, gtab, ctab):
    mesh = plsc.VectorSubcoreMesh(
        core_axis_name="c", subcore_axis_name="s",
        num_cores=_NC, num_subcores=_NS)

    @functools.partial(
        pl.kernel,
        out_type=[jax.ShapeDtypeStruct((_BCH, DIM), jnp.float32)] * 4,
        mesh=mesh,
        scratch_types=[
            pltpu.VMEM((4, _ROWS_PER_W), jnp.int32),
            [pltpu.VMEM((_CHUNK, DIM), jnp.float32)] * (4 * _NCHUNK),
            [pltpu.SemaphoreType.DMA] * 4,
            [pltpu.SemaphoreType.DMA] * (4 * _NCHUNK),
            [pltpu.SemaphoreType.DMA] * (4 * _NCHUNK),
        ],
    )
    def gather_kernel(uid_h, iid_h, gid_h, cid_h, ut_h, it_h, gt_h, ct_h,
                      o0, o1, o2, o3, idx_v, bufs, isems, gsems, wsems):
        wid = lax.axis_index("s") * _NC + lax.axis_index("c")
        base = wid * _ROWS_PER_W
        idx_hs = (uid_h, iid_h, gid_h, cid_h)
        tab_hs = (ut_h, it_h, gt_h, ct_h)
        out_hs = (o0, o1, o2, o3)
        ic = [pltpu.async_copy(
                  idx_hs[t].at[pl.ds(k * _BCH + base, _ROWS_PER_W)],
                  idx_v.at[t], isems[t]) for t in range(4)]
        g = [None] * (4 * _NCHUNK)
        wb = [None] * (4 * _NCHUNK)
        for t in range(4):
            ic[t].wait()
            for c in range(_NCHUNK):
                i = t * _NCHUNK + c
                g[i] = pltpu.async_copy(
                    tab_hs[t].at[idx_v.at[t, pl.ds(c * _CHUNK, _CHUNK)]],
                    bufs[i], gsems[i])
        for t in range(4):
            for c in range(_NCHUNK):
                i = t * _NCHUNK + c
                g[i].wait()
                wb[i] = pltpu.async_copy(
                    bufs[i],
                    out_hs[t].at[pl.ds(base + c * _CHUNK, _CHUNK)],
                    wsems[i])
        for i in range(4 * _NCHUNK):
            wb[i].wait()

    return gather_kernel(uid, iid, gid, cid, utab, itab, gtab, ctab)


_TB = 2048
_TB_OUT = 4096


def _moments_body(e0, e1, e2, e3, w1, hin, s_out, m_out, h_out):
    i = pl.program_id(0)
    h = (jnp.dot(e0[...], w1[0:128, :], preferred_element_type=jnp.float32)
         + jnp.dot(e1[...], w1[128:256, :], preferred_element_type=jnp.float32)
         + jnp.dot(e2[...], w1[256:384, :], preferred_element_type=jnp.float32)
         + jnp.dot(e3[...], w1[384:512, :], preferred_element_type=jnp.float32))

    @pl.when(i == 0)
    def _init():
        s_out[...] = jnp.zeros_like(s_out)
        m_out[...] = jnp.zeros_like(m_out)

    s_out[...] += jnp.sum(h, axis=0, keepdims=True)
    m_out[...] += lax.dot_general(h, h, (((0,), (0,)), ((), ())),
                                  preferred_element_type=jnp.float32)
    h_out[...] = h.astype(jnp.bfloat16)


def _moments(k, e0, e1, e2, e3, w1, hbuf):
    grid = (_BCH // _TB,)
    nblk = _BCH // _TB
    eb = pl.BlockSpec((_TB, DIM), lambda i: (i, 0))
    body = _moments_body if k > 0 else (
        lambda e0, e1, e2, e3, w1, s, m, h:
            _moments_body(e0, e1, e2, e3, w1, None, s, m, h))
    return pl.pallas_call(
        body,
        grid=grid,
        in_specs=[eb, eb, eb, eb, pl.BlockSpec((512, 256), lambda i: (0, 0))] +
                 ([pl.BlockSpec(memory_space=pltpu.MemorySpace.HBM)]
                  if k > 0 else []),
        out_specs=[pl.BlockSpec((1, 256), lambda i: (0, 0)),
                   pl.BlockSpec((256, 256), lambda i: (0, 0)),
                   pl.BlockSpec((_TB, 256), lambda i: (k * nblk + i, 0))],
        out_shape=[jax.ShapeDtypeStruct((1, 256), jnp.float32),
                   jax.ShapeDtypeStruct((256, 256), jnp.float32),
                   jax.ShapeDtypeStruct((BATCH, 256), jnp.bfloat16)],
        input_output_aliases={5: 2} if k > 0 else {},
    )(e0, e1, e2, e3, w1, *([hbuf] if k > 0 else []))


def _col(row):
    k = row.shape[1]
    b = jnp.broadcast_to(row, (k, k))
    rows_i = lax.broadcasted_iota(jnp.int32, (k, k), 0)
    cols_i = lax.broadcasted_iota(jnp.int32, (k, k), 1)
    return jnp.sum(jnp.where(rows_i == cols_i, b, 0.0), axis=1, keepdims=True)


def _chain_math(s1, m1, g1, b1, w2, g2, b2, w3, g3, b3):
    n = float(BATCH)
    mu1 = s1 / n
    m1n = m1 / n
    mu1c = _col(mu1)
    cov1 = m1n - mu1c * mu1
    rows_i = lax.broadcasted_iota(jnp.int32, (256, 256), 0)
    cols_i = lax.broadcasted_iota(jnp.int32, (256, 256), 1)
    var1 = jnp.sum(jnp.where(rows_i == cols_i, cov1, 0.0),
                   axis=0, keepdims=True)
    a1 = g1 * lax.rsqrt(var1 + EPS)
    c1 = b1 - mu1 * a1

    a1c = _col(a1)
    cov1s = cov1 * (a1c * a1)
    t2 = jnp.dot(cov1s, w2, preferred_element_type=jnp.float32)
    var2 = jnp.sum(w2 * t2, axis=0, keepdims=True)
    mu2 = jnp.dot(mu1 * a1 + c1, w2,
                  preferred_element_type=jnp.float32)
    a2 = g2 * lax.rsqrt(var2 + EPS)
    c2 = b2 - mu2 * a2

    cov2 = lax.dot_general(w2, t2, (((0,), (0,)), ((), ())),
                           preferred_element_type=jnp.float32)
    a2c = _col(a2)
    cov2s = cov2 * (a2c * a2)
    t3 = jnp.dot(cov2s, w3, preferred_element_type=jnp.float32)
    var3 = jnp.sum(w3 * t3, axis=0, keepdims=True)
    mu3 = jnp.dot(mu2 * a2 + c2, w3,
                  preferred_element_type=jnp.float32)
    a3 = g3 * lax.rsqrt(var3 + EPS)
    c3 = b3 - mu3 * a3

    r12 = (a1c * w2) * a2
    r = jnp.dot(r12, w3,
                preferred_element_type=jnp.float32) * a3
    q12 = jnp.dot(c1, w2, preferred_element_type=jnp.float32) * a2 + c2
    q = jnp.dot(q12, w3, preferred_element_type=jnp.float32) * a3 + c3
    return r, _col(q)


def _output_body(*refs):
    h1 = refs[0]
    s_refs = refs[1:1 + _NSPLIT]
    m_refs = refs[1 + _NSPLIT:1 + 2 * _NSPLIT]
    (g1, b1, w2, g2, b2, w3, g3, b3, o, r_v, q_v) = refs[1 + 2 * _NSPLIT:]
    i = pl.program_id(0)

    @pl.when(i == 0)
    def _fold():
        s1 = sum(ref[...] for ref in s_refs)
        m1 = sum(ref[...] for ref in m_refs)
        r, qt = _chain_math(s1, m1, g1[...], b1[...], w2[...],
                            g2[...], b2[...], w3[...], g3[...], b3[...])
        r_v[...] = r
        q_v[...] = qt

    acc = lax.dot_general(r_v[...], h1[...].astype(jnp.float32),
                          (((0,), (1,)), ((), ())),
                          preferred_element_type=jnp.float32)
    o[...] = jax.nn.sigmoid(acc + q_v[...])


def _output(h1, s_parts, m_parts, g1, b1, w2, g2, b2, w3, g3, b3):
    grid = (BATCH // _TB_OUT,)
    full = lambda shape: pl.BlockSpec(shape, lambda i: (0, 0))
    return pl.pallas_call(
        _output_body,
        grid=grid,
        in_specs=[pl.BlockSpec((_TB_OUT, 256), lambda i: (i, 0))] +
                 [full((1, 256))] * _NSPLIT + [full((256, 256))] * _NSPLIT +
                 [full((1, 256)), full((1, 256)), full((256, 128)),
                  full((1, 128)), full((1, 128)), full((128, 2)),
                  full((1, 2)), full((1, 2))],
        out_specs=pl.BlockSpec((2, _TB_OUT), lambda i: (0, i)),
        out_shape=jax.ShapeDtypeStruct((2, BATCH), jnp.float32),
        scratch_shapes=[pltpu.VMEM((256, 2), jnp.float32),
                        pltpu.VMEM((2, 1), jnp.float32)],
    )(h1, *s_parts, *m_parts, g1, b1, w2, g2, b2, w3, g3, b3)


def kernel(user_id, item_id, user_geohash, item_cate, label,
           user_tab, item_tab, geo_tab, cate_tab,
           W1, g1, b1, W2, g2, b2, W3, g3, b3):
    s_parts, m_parts = [], []
    hbuf = None
    for k in range(_NSPLIT):
        e0, e1, e2, e3 = _sc_gather(k, user_id, item_id, user_geohash,
                                    item_cate, user_tab, item_tab,
                                    geo_tab, cate_tab)
        s1, m1, hbuf = _moments(k, e0, e1, e2, e3, W1, hbuf)
        s_parts.append(s1)
        m_parts.append(m1)
    return jnp.transpose(_output(
        hbuf, s_parts, m_parts,
        g1.reshape(1, 256), b1.reshape(1, 256), W2,
        g2.reshape(1, 128), b2.reshape(1, 128), W3,
        g3.reshape(1, 2), b3.reshape(1, 2)))
